# baseline (speedup 1.0000x reference)
import functools

import jax
import jax.numpy as jnp
import numpy as np
from jax import lax
from jax.experimental import pallas as pl
from jax.experimental.pallas import tpu as pltpu
from jax.experimental.pallas import tpu_sc as plsc

_N = 10000
_NP = 10112
_E = 320000
_NW = 32
_EPW = _E // _NW
_C = 2000
_NCH = _EPW // _C

_TWO_PI = float(2.0 * np.pi)
_INV_2PI = float(1.0 / (2.0 * np.pi))
_COS_C = (1.0, -1.0 / 2, 1.0 / 24, -1.0 / 720, 1.0 / 40320,
          -1.0 / 3628800, 1.0 / 479001600, -1.0 / 87178291200)
_RND = 12582912.0


def _leaky(x):
    return jnp.where(x > 0, x, 0.01 * x)


def _dotT(lhs, rhs, lhs_dim, rhs_dim):
    return lax.dot_general(lhs, rhs, (((lhs_dim,), (rhs_dim,)), ((), ())),
                           preferred_element_type=jnp.float32)


def _node_body(xue_ref, xap_ref, Wnu1, bnu1, Wnu2, bnu2, Wna1, bna1,
               Wna2, bna2, qcW, hueT_ref,
               su0_ref, su1_ref, su2_ref, du0_ref, du1_ref, du2_ref):
    qc_su = qcW[2:4, :]
    qc_du = qcW[4:6, :]
    a1 = _leaky(_dotT(Wnu1[...], xue_ref[...], 0, 1)
                + bnu1[...].reshape(-1, 1))
    hueT = _dotT(Wnu2[...], a1, 0, 0) + bnu2[...].reshape(-1, 1)
    hueT_ref[...] = hueT
    duT = _dotT(qc_du, hueT, 0, 0)
    a2 = _leaky(_dotT(Wna1[...], xap_ref[...], 0, 1)
                + bna1[...].reshape(-1, 1))
    hapT = _dotT(Wna2[...], a2, 0, 0) + bna2[...].reshape(-1, 1)
    suT = _dotT(qc_su, hapT, 0, 0)
    su0_ref[...] = suT[0]
    su1_ref[...] = suT[1]
    su2_ref[...] = suT[2]
    du0_ref[...] = duT[0]
    du1_ref[...] = duT[1]
    du2_ref[...] = duT[2]


def _edge_body(eaT_ref, We1, be1, We2, be2, qcW, qcb,
               ec0_ref, ec1_ref, ec2_ref):
    qc_e = qcW[0:2, :]
    h = _leaky(_dotT(We1[...], eaT_ref[...], 0, 0)
               + be1[...].reshape(-1, 1))
    We23T = _dotT(qc_e, We2[...], 0, 1)
    be3T = (_dotT(qc_e, be2[...].reshape(1, -1), 0, 1)
            + qcb[...].reshape(-1, 1))
    ecT = jnp.dot(We23T, h,
                  preferred_element_type=jnp.float32) + be3T
    ec0_ref[...] = ecT[0]
    ec1_ref[...] = ecT[1]
    ec2_ref[...] = ecT[2]


def _sc_body(su0_hbm, su1_hbm, su2_hbm, du0_hbm, du1_hbm, du2_hbm,
             src_hbm, dst_hbm, ec0_hbm, ec1_hbm, ec2_hbm,
             out_hbm, su_t, du_t, acc, srcb, dstb, ecb, sem_t, sem0, sem1):
    wid = lax.axis_index("s") * 2 + lax.axis_index("c")

    h_t = []
    for j, ref in enumerate((su0_hbm, su1_hbm, su2_hbm)):
        h_t.append(pltpu.async_copy(ref, su_t.at[pl.ds(j * _N, _N)], sem_t))
    for j, ref in enumerate((du0_hbm, du1_hbm, du2_hbm)):
        h_t.append(pltpu.async_copy(ref, du_t.at[pl.ds(j * _N, _N)], sem_t))

    sems = (sem0, sem1)

    def _start_chunk(ch):
        b = ch % 2
        base = wid * _EPW + ch * _C
        hs = pltpu.async_copy(src_hbm.at[pl.ds(base, _C)],
                              srcb.at[pl.ds(b * _C, _C)], sems[b])
        hd = pltpu.async_copy(dst_hbm.at[pl.ds(base, _C)],
                              dstb.at[pl.ds(b * _C, _C)], sems[b])
        he = tuple(
            pltpu.async_copy(ec_hbm.at[pl.ds(base, _C)],
                             ecb.at[pl.ds((b * 3 + j) * _C, _C)], sems[b])
            for j, ec_hbm in enumerate((ec0_hbm, ec1_hbm, ec2_hbm)))
        return (hs, hd) + he

    pend = _start_chunk(0)

    zero16 = jnp.zeros((16,), jnp.float32)

    @plsc.parallel_loop(0, (_NP * 4) // 16, unroll=8)
    def _zbody(i):
        acc[pl.ds(i * 16, 16)] = zero16

    for h in h_t:
        h.wait()

    ones16 = jnp.full((16,), 1.0, jnp.float32)

    for ch in range(_NCH):
        b = ch % 2
        for h in pend:
            h.wait()
        if ch + 1 < _NCH:
            pend = _start_chunk(ch + 1)
        soff = b * _C
        eoff = b * 3 * _C

        @plsc.parallel_loop(0, _C // 16, unroll=5)
        def _gbody(g):
            g16 = g * 16
            rs = srcb[pl.ds(soff + g16, 16)]
            rd = dstb[pl.ds(soff + g16, 16)]
            for j in range(3):
                sj = plsc.load_gather(su_t, [rs + j * _N])
                dj = plsc.load_gather(du_t, [rd + j * _N])
                ej = ecb[pl.ds(eoff + j * _C + g16, 16)]
                x = ej + sj + dj
                kf = (x * _INV_2PI + _RND) - _RND
                r = x - kf * _TWO_PI
                y = r * r
                pv = jnp.full((16,), _COS_C[7], jnp.float32)
                for c in (_COS_C[6], _COS_C[5], _COS_C[4], _COS_C[3],
                          _COS_C[2], _COS_C[1], _COS_C[0]):
                    pv = pv * y + c
                plsc.addupdate_scatter(acc, [rd + j * _NP], pv)
            plsc.addupdate_scatter(acc, [rd + 3 * _NP], ones16)

    pltpu.sync_copy(acc, out_hbm.at[pl.ds(wid * (4 * _NP), 4 * _NP)])


def _sc_edges(su_planes, du_planes, src, dst, ec_planes):
    run = functools.partial(
        pl.kernel,
        out_type=jax.ShapeDtypeStruct((_NW * 4 * _NP,), jnp.float32),
        mesh=plsc.VectorSubcoreMesh(core_axis_name="c", subcore_axis_name="s",
                                    num_cores=2, num_subcores=16),
        compiler_params=pltpu.CompilerParams(needs_layout_passes=False),
        scratch_types=[
            pltpu.VMEM((_N * 3,), jnp.float32),
            pltpu.VMEM((_N * 3,), jnp.float32),
            pltpu.VMEM((_NP * 4,), jnp.float32),
            pltpu.VMEM((2 * _C,), jnp.int32),
            pltpu.VMEM((2 * _C,), jnp.int32),
            pltpu.VMEM((2 * 3 * _C,), jnp.float32),
            pltpu.SemaphoreType.DMA,
            pltpu.SemaphoreType.DMA,
            pltpu.SemaphoreType.DMA,
        ],
    )(_sc_body)
    return run(*su_planes, *du_planes, src, dst, *ec_planes)


def _post_body(parts_ref, hueT_ref, Wu1, bu1, Wu2, bu2,
               lng, lnb, Wf1, bf1, Wf2, bf2, Wf3, bf3, out_ref):
    def _plane_sum(j):
        s = parts_ref[pl.ds(j * _NP, _N)]
        for w in range(1, _NW):
            s = s + parts_ref[pl.ds(w * 4 * _NP + j * _NP, _N)]
        return s.reshape(1, _N)

    m0 = _plane_sum(0)
    m1 = _plane_sum(1)
    m2 = _plane_sum(2)
    deg = _plane_sum(3)
    inv = 1.0 / jnp.maximum(deg, 1.0)
    hueT = hueT_ref[...]
    xin = jnp.concatenate([hueT, m0 * inv, m1 * inv, m2 * inv], axis=0)
    u = _leaky(_dotT(Wu1[...], xin, 0, 0) + bu1[...].reshape(-1, 1))
    upd = _dotT(Wu2[...], u, 0, 0) + bu2[...].reshape(-1, 1)
    mask = (deg > 0).astype(jnp.float32)
    h = hueT + upd * mask
    diff = (h[0:1, :] - h[1:2, :]) * 0.5
    t = diff * lax.rsqrt(diff * diff + 1e-5)
    lngv = lng[...]
    lngs = jnp.concatenate([lngv[0:1], -lngv[1:2]], axis=0)
    WtT = _dotT(Wf1[...], lngs.reshape(1, -1), 0, 1)
    btT = (_dotT(Wf1[...], lnb[...].reshape(1, -1), 0, 1)
           + bf1[...].reshape(-1, 1))
    f = _leaky(t * WtT + btT)
    f = _leaky(_dotT(Wf2[...], f, 0, 0) + bf2[...].reshape(-1, 1))
    o = _dotT(Wf3[...], f, 0, 0) + bf3[...].reshape(-1, 1)
    out_ref[...] = jax.nn.sigmoid(o)


def kernel(x_UE, x_AP, edge_attr, edge_index, batch, params):
    p = params
    f32 = jnp.float32

    nplane = jax.ShapeDtypeStruct((_N,), f32)
    hueT, su0, su1, su2, du0, du1, du2 = pl.pallas_call(
        _node_body,
        out_shape=(jax.ShapeDtypeStruct((2, _N), f32),) + (nplane,) * 6,
    )(x_UE, x_AP,
      p["W_nu1"], p["b_nu1"], p["W_nu2"], p["b_nu2"],
      p["W_na1"], p["b_na1"], p["W_na2"], p["b_na2"], p["qc_W"])

    be = 32768
    eplane = jax.ShapeDtypeStruct((10 * be,), f32)
    ec0, ec1, ec2 = pl.pallas_call(
        _edge_body,
        grid=(10,),
        in_specs=[
            pl.BlockSpec((16, be), lambda i: (0, i)),
            pl.BlockSpec((16, 64), lambda i: (0, 0)),
            pl.BlockSpec((64,), lambda i: (0,)),
            pl.BlockSpec((64, 2), lambda i: (0, 0)),
            pl.BlockSpec((2,), lambda i: (0,)),
            pl.BlockSpec((6, 3), lambda i: (0, 0)),
            pl.BlockSpec((3,), lambda i: (0,)),
        ],
        out_specs=(pl.BlockSpec((be,), lambda i: (i,)),
                   pl.BlockSpec((be,), lambda i: (i,)),
                   pl.BlockSpec((be,), lambda i: (i,))),
        out_shape=(eplane, eplane, eplane),
    )(edge_attr.T, p["W_e1"], p["b_e1"], p["W_e2"], p["b_e2"],
      p["qc_W"], p["qc_b"])

    src = edge_index[0].astype(jnp.int32)
    dst = edge_index[1].astype(jnp.int32)
    parts = _sc_edges((su0, su1, su2), (du0, du1, du2), src, dst,
                      (ec0, ec1, ec2))

    outT = pl.pallas_call(
        _post_body,
        out_shape=jax.ShapeDtypeStruct((2, _N), f32),
    )(parts, hueT,
      p["W_u1"], p["b_u1"], p["W_u2"], p["b_u2"],
      p["ln_g"], p["ln_b"], p["W_f1"], p["b_f1"],
      p["W_f2"], p["b_f2"], p["W_f3"], p["b_f3"])
    return outT.T


# speedup vs baseline: 1.0115x; 1.0115x over previous
import functools

import jax
import jax.numpy as jnp
import numpy as np
from jax import lax
from jax.experimental import pallas as pl
from jax.experimental.pallas import tpu as pltpu
from jax.experimental.pallas import tpu_sc as plsc

_N = 10000
_NP = 10112
_E = 320000
_NW = 32
_EPW = _E // _NW
_C = 2000
_NCH = _EPW // _C

_TWO_PI = float(2.0 * np.pi)
_INV_2PI = float(1.0 / (2.0 * np.pi))
_COS_C = (1.0, -1.0 / 2, 1.0 / 24, -1.0 / 720, 1.0 / 40320,
          -1.0 / 3628800, 1.0 / 479001600, -1.0 / 87178291200)
_RND = 12582912.0


def _leaky(x):
    return jnp.where(x > 0, x, 0.01 * x)


def _dotT(lhs, rhs, lhs_dim, rhs_dim):
    return lax.dot_general(lhs, rhs, (((lhs_dim,), (rhs_dim,)), ((), ())),
                           preferred_element_type=jnp.float32)


def _node_body(xue_ref, xap_ref, Wnu1, bnu1, Wnu2, bnu2, Wna1, bna1,
               Wna2, bna2, qcW, hueT_ref,
               su0_ref, su1_ref, su2_ref, du0_ref, du1_ref, du2_ref):
    qc_su = qcW[2:4, :]
    qc_du = qcW[4:6, :]
    a1 = _leaky(_dotT(Wnu1[...], xue_ref[...], 0, 1)
                + bnu1[...].reshape(-1, 1))
    hueT = _dotT(Wnu2[...], a1, 0, 0) + bnu2[...].reshape(-1, 1)
    hueT_ref[...] = hueT
    duT = _dotT(qc_du, hueT, 0, 0)
    a2 = _leaky(_dotT(Wna1[...], xap_ref[...], 0, 1)
                + bna1[...].reshape(-1, 1))
    hapT = _dotT(Wna2[...], a2, 0, 0) + bna2[...].reshape(-1, 1)
    suT = _dotT(qc_su, hapT, 0, 0)
    su0_ref[...] = suT[0]
    su1_ref[...] = suT[1]
    su2_ref[...] = suT[2]
    du0_ref[...] = duT[0]
    du1_ref[...] = duT[1]
    du2_ref[...] = duT[2]


def _edge_body(eaT_ref, We1, be1, We2, be2, qcW, qcb,
               ec0_ref, ec1_ref, ec2_ref):
    qc_e = qcW[0:2, :]
    h = _leaky(_dotT(We1[...], eaT_ref[...], 0, 0)
               + be1[...].reshape(-1, 1))
    We23T = _dotT(qc_e, We2[...], 0, 1)
    be3T = (_dotT(qc_e, be2[...].reshape(1, -1), 0, 1)
            + qcb[...].reshape(-1, 1))
    ecT = jnp.dot(We23T, h,
                  preferred_element_type=jnp.float32) + be3T
    ec0_ref[...] = ecT[0]
    ec1_ref[...] = ecT[1]
    ec2_ref[...] = ecT[2]


def _sc_body(su0_hbm, su1_hbm, su2_hbm, du0_hbm, du1_hbm, du2_hbm,
             src_hbm, dst_hbm, ec0_hbm, ec1_hbm, ec2_hbm,
             out_hbm, su_t, du_t, acc, srcb, dstb, ecb, sem_t, sem0, sem1):
    wid = lax.axis_index("s") * 2 + lax.axis_index("c")

    h_t = []
    for j, ref in enumerate((su0_hbm, su1_hbm, su2_hbm)):
        h_t.append(pltpu.async_copy(ref, su_t.at[pl.ds(j * _N, _N)], sem_t))
    for j, ref in enumerate((du0_hbm, du1_hbm, du2_hbm)):
        h_t.append(pltpu.async_copy(ref, du_t.at[pl.ds(j * _N, _N)], sem_t))

    sems = (sem0, sem1)

    def _start_chunk(ch):
        b = ch % 2
        base = wid * _EPW + ch * _C
        hs = pltpu.async_copy(src_hbm.at[pl.ds(base, _C)],
                              srcb.at[pl.ds(b * _C, _C)], sems[b])
        hd = pltpu.async_copy(dst_hbm.at[pl.ds(base, _C)],
                              dstb.at[pl.ds(b * _C, _C)], sems[b])
        he = tuple(
            pltpu.async_copy(ec_hbm.at[pl.ds(base, _C)],
                             ecb.at[pl.ds((b * 3 + j) * _C, _C)], sems[b])
            for j, ec_hbm in enumerate((ec0_hbm, ec1_hbm, ec2_hbm)))
        return (hs, hd) + he

    pend = _start_chunk(0)

    zero16 = jnp.zeros((16,), jnp.float32)

    @plsc.parallel_loop(0, (_NP * 4) // 16, unroll=8)
    def _zbody(i):
        acc[pl.ds(i * 16, 16)] = zero16

    for h in h_t:
        h.wait()

    ones16 = jnp.full((16,), 1.0, jnp.float32)

    for ch in range(_NCH):
        b = ch % 2
        for h in pend:
            h.wait()
        if ch + 1 < _NCH:
            pend = _start_chunk(ch + 1)
        soff = b * _C
        eoff = b * 3 * _C

        @plsc.parallel_loop(0, _C // 16, unroll=4)
        def _gbody(g):
            g16 = g * 16
            rs = srcb[pl.ds(soff + g16, 16)]
            rd = dstb[pl.ds(soff + g16, 16)]
            for j in range(3):
                sj = plsc.load_gather(su_t, [rs + j * _N])
                dj = plsc.load_gather(du_t, [rd + j * _N])
                ej = ecb[pl.ds(eoff + j * _C + g16, 16)]
                x = ej + sj + dj
                kf = (x * _INV_2PI + _RND) - _RND
                r = x - kf * _TWO_PI
                y = r * r
                pv = jnp.full((16,), _COS_C[7], jnp.float32)
                for c in (_COS_C[6], _COS_C[5], _COS_C[4], _COS_C[3],
                          _COS_C[2], _COS_C[1], _COS_C[0]):
                    pv = pv * y + c
                plsc.addupdate_scatter(acc, [rd + j * _NP], pv)
            plsc.addupdate_scatter(acc, [rd + 3 * _NP], ones16)

    pltpu.sync_copy(acc, out_hbm.at[pl.ds(wid * (4 * _NP), 4 * _NP)])


def _sc_edges(su_planes, du_planes, src, dst, ec_planes):
    run = functools.partial(
        pl.kernel,
        out_type=jax.ShapeDtypeStruct((_NW * 4 * _NP,), jnp.float32),
        mesh=plsc.VectorSubcoreMesh(core_axis_name="c", subcore_axis_name="s",
                                    num_cores=2, num_subcores=16),
        compiler_params=pltpu.CompilerParams(needs_layout_passes=False),
        scratch_types=[
            pltpu.VMEM((_N * 3,), jnp.float32),
            pltpu.VMEM((_N * 3,), jnp.float32),
            pltpu.VMEM((_NP * 4,), jnp.float32),
            pltpu.VMEM((2 * _C,), jnp.int32),
            pltpu.VMEM((2 * _C,), jnp.int32),
            pltpu.VMEM((2 * 3 * _C,), jnp.float32),
            pltpu.SemaphoreType.DMA,
            pltpu.SemaphoreType.DMA,
            pltpu.SemaphoreType.DMA,
        ],
    )(_sc_body)
    return run(*su_planes, *du_planes, src, dst, *ec_planes)


def _post_body(parts_ref, hueT_ref, Wu1, bu1, Wu2, bu2,
               lng, lnb, Wf1, bf1, Wf2, bf2, Wf3, bf3, out_ref):
    def _plane_sum(j):
        s = parts_ref[pl.ds(j * _NP, _N)]
        for w in range(1, _NW):
            s = s + parts_ref[pl.ds(w * 4 * _NP + j * _NP, _N)]
        return s.reshape(1, _N)

    m0 = _plane_sum(0)
    m1 = _plane_sum(1)
    m2 = _plane_sum(2)
    deg = _plane_sum(3)
    inv = 1.0 / jnp.maximum(deg, 1.0)
    hueT = hueT_ref[...]
    xin = jnp.concatenate([hueT, m0 * inv, m1 * inv, m2 * inv], axis=0)
    u = _leaky(_dotT(Wu1[...], xin, 0, 0) + bu1[...].reshape(-1, 1))
    upd = _dotT(Wu2[...], u, 0, 0) + bu2[...].reshape(-1, 1)
    mask = (deg > 0).astype(jnp.float32)
    h = hueT + upd * mask
    diff = (h[0:1, :] - h[1:2, :]) * 0.5
    t = diff * lax.rsqrt(diff * diff + 1e-5)
    lngv = lng[...]
    lngs = jnp.concatenate([lngv[0:1], -lngv[1:2]], axis=0)
    WtT = _dotT(Wf1[...], lngs.reshape(1, -1), 0, 1)
    btT = (_dotT(Wf1[...], lnb[...].reshape(1, -1), 0, 1)
           + bf1[...].reshape(-1, 1))
    f = _leaky(t * WtT + btT)
    f = _leaky(_dotT(Wf2[...], f, 0, 0) + bf2[...].reshape(-1, 1))
    o = _dotT(Wf3[...], f, 0, 0) + bf3[...].reshape(-1, 1)
    out_ref[...] = jax.nn.sigmoid(o)


def kernel(x_UE, x_AP, edge_attr, edge_index, batch, params):
    p = params
    f32 = jnp.float32

    nplane = jax.ShapeDtypeStruct((_N,), f32)
    hueT, su0, su1, su2, du0, du1, du2 = pl.pallas_call(
        _node_body,
        out_shape=(jax.ShapeDtypeStruct((2, _N), f32),) + (nplane,) * 6,
    )(x_UE, x_AP,
      p["W_nu1"], p["b_nu1"], p["W_nu2"], p["b_nu2"],
      p["W_na1"], p["b_na1"], p["W_na2"], p["b_na2"], p["qc_W"])

    be = 32768
    eplane = jax.ShapeDtypeStruct((10 * be,), f32)
    ec0, ec1, ec2 = pl.pallas_call(
        _edge_body,
        grid=(10,),
        in_specs=[
            pl.BlockSpec((16, be), lambda i: (0, i)),
            pl.BlockSpec((16, 64), lambda i: (0, 0)),
            pl.BlockSpec((64,), lambda i: (0,)),
            pl.BlockSpec((64, 2), lambda i: (0, 0)),
            pl.BlockSpec((2,), lambda i: (0,)),
            pl.BlockSpec((6, 3), lambda i: (0, 0)),
            pl.BlockSpec((3,), lambda i: (0,)),
        ],
        out_specs=(pl.BlockSpec((be,), lambda i: (i,)),
                   pl.BlockSpec((be,), lambda i: (i,)),
                   pl.BlockSpec((be,), lambda i: (i,))),
        out_shape=(eplane, eplane, eplane),
    )(edge_attr.T, p["W_e1"], p["b_e1"], p["W_e2"], p["b_e2"],
      p["qc_W"], p["qc_b"])

    src = edge_index[0].astype(jnp.int32)
    dst = edge_index[1].astype(jnp.int32)
    parts = _sc_edges((su0, su1, su2), (du0, du1, du2), src, dst,
                      (ec0, ec1, ec2))

    outT = pl.pallas_call(
        _post_body,
        out_shape=jax.ShapeDtypeStruct((2, _N), f32),
    )(parts, hueT,
      p["W_u1"], p["b_u1"], p["W_u2"], p["b_u2"],
      p["ln_g"], p["ln_b"], p["W_f1"], p["b_f1"],
      p["W_f2"], p["b_f2"], p["W_f3"], p["b_f3"])
    return outT.T
